# Initial kernel scaffold; baseline (speedup 1.0000x reference)
import jax
import jax.numpy as jnp
from jax.experimental import pallas as pl


def kernel(obj_dist, object_feature, rel_ind, pred_dist, obj_table, pred_table, W_v, b_v, W_oe, b_oe, W_pe, b_pe, W_e, b_e, W_n, W_a, b_n):
    raise NotImplementedError("write your pallas kernel here")


# speedup vs baseline: 2238.3114x; 2238.3114x over previous
import functools

import jax
import jax.numpy as jnp
from jax import lax
from jax.experimental import pallas as pl
from jax.experimental.pallas import tpu as pltpu
from jax.experimental.pallas import tpu_sc as plsc

B, N, K, L, OFS = 8, 1250, 20000, 128, 512
N_OBJ, N_PRED = 150, 50

NC, NS = 2, 16
BPC = B // NC
WPB = NS // BPC
EPW = K // WPB
CH = 100
NCH = EPW // CH
CHB = K // CH
NPAD = 1280
ZROWS = BPC * NPAD // NS

R = B * N
BR = 400
BE = 2000

_f32 = jnp.float32


def _dot(a, b):
    return jnp.dot(a, b, preferred_element_type=_f32)


def _fuse_body(idx_ref, of_ref, otab_ref, woe_ref, wv_ref, bias0_ref,
               wea_ref, wec_ref, x_ref, p1_ref, p3_ref):
    ot = _dot(otab_ref[...], woe_ref[...])
    iot = lax.broadcasted_iota(jnp.int32, (BR, N_OBJ), 1)
    oh = (idx_ref[...] == iot).astype(_f32)
    emb = _dot(oh, ot)
    x = jnp.maximum(_dot(of_ref[...], wv_ref[...]) + emb + bias0_ref[...], 0.0)
    x_ref[...] = x
    p1_ref[...] = _dot(x, wea_ref[...])
    p3_ref[...] = _dot(x, wec_ref[...])


def _fuse(idx2, of2, obj_table, W_oe, W_v, bias0, Wea, Wec):
    full = lambda s: pl.BlockSpec(s, lambda i: (0, 0))
    return pl.pallas_call(
        _fuse_body,
        grid=(R // BR,),
        in_specs=[
            pl.BlockSpec((BR, 1), lambda i: (i, 0)),
            pl.BlockSpec((BR, OFS), lambda i: (i, 0)),
            full((N_OBJ, L)), full((L, L)), full((OFS, L)), full((1, L)),
            full((L, L)), full((L, L)),
        ],
        out_specs=[pl.BlockSpec((BR, L), lambda i: (i, 0))] * 3,
        out_shape=[jax.ShapeDtypeStruct((R, L), _f32)] * 3,
    )(idx2, of2, obj_table, W_oe, W_v, bias0, Wea, Wec)


def _t0_body(pt_ref, wpe_ref, bpe_ref, web_ref, be0_ref, t0_ref):
    p = _dot(pt_ref[...], wpe_ref[...]) + bpe_ref[...]
    t0_ref[...] = _dot(p, web_ref[...]) + be0_ref[...]


def _t0(pred_table, W_pe, bpe2, We0b, be02):
    return pl.pallas_call(
        _t0_body,
        out_shape=jax.ShapeDtypeStruct((N_PRED, L), _f32),
    )(pred_table, W_pe, bpe2, We0b, be02)


def _mm_body(m_ref, w_ref, b_ref, o_ref):
    o_ref[...] = _dot(m_ref[...], w_ref[...]) + b_ref[...]


def _mm(m2, w, b2):
    full = lambda s: pl.BlockSpec(s, lambda i: (0, 0))
    return pl.pallas_call(
        _mm_body,
        grid=(B * K // BE,),
        in_specs=[pl.BlockSpec((BE, L), lambda i: (i, 0)), full((L, L)),
                  full((1, L))],
        out_specs=pl.BlockSpec((BE, L), lambda i: (i, 0)),
        out_shape=jax.ShapeDtypeStruct((B * K, L), _f32),
    )(m2, w, b2)


NW = NC * NS


def _node_body(x_ref, agg_ref, wn_ref, wa_ref, bn_ref, *rest):
    x = x_ref[...]
    h = _dot(x, wn_ref[...]) + _dot(agg_ref[...], wa_ref[...]) + bn_ref[...]
    xn = x + jnp.maximum(h, 0.0)
    if len(rest) == 1:
        rest[0][...] = xn
    else:
        wea_ref, wec_ref, xo_ref, p1_ref, p3_ref = rest
        xo_ref[...] = xn
        p1_ref[...] = _dot(xn, wea_ref[...])
        p3_ref[...] = _dot(xn, wec_ref[...])


def _node(x, agg, Wn, Wa, bn2, Wea=None, Wec=None):
    full = lambda s: pl.BlockSpec(s, lambda i: (0, 0))
    blk = pl.BlockSpec((BR, L), lambda i: (i, 0))
    n_out = 1 if Wea is None else 3
    ins = [x, agg, Wn, Wa, bn2] + ([] if Wea is None else [Wea, Wec])
    return pl.pallas_call(
        _node_body,
        grid=(R // BR,),
        in_specs=[blk, blk, full((L, L)), full((L, L)), full((1, L))]
        + [full((L, L))] * (n_out - 1),
        out_specs=[blk] * n_out,
        out_shape=[jax.ShapeDtypeStruct((R, L), _f32)] * n_out,
    )(*ins)


def _sc_edge_inner(q1_h, q3_h, subg_h, obg_h, subl_h, obl_h, pd_h, t0_h, m_h,
                   msg_h, agg_h, subg_v, obg_v, subl_v, obl_v, pd_v,
                   g1, g3, gp, mo, aggs, s1, s3, sp, so):
    c = lax.axis_index("c")
    s = lax.axis_index("s")
    b = c * BPC + s // WPB
    q = s % WPB
    w = b * WPB + q
    base = s * ZROWS

    def zrow(r, _):
        for li in range(L // 16):
            g1[r, pl.ds(li * 16, 16)] = jnp.zeros((16,), _f32)
        return 0
    lax.fori_loop(0, CH, zrow, 0)
    for t in range(ZROWS // 80):
        pltpu.sync_copy(g1.at[pl.ds(0, 80)], aggs.at[pl.ds(base + t * 80, 80)])

    pltpu.sync_copy(subg_h.at[w], subg_v)
    pltpu.sync_copy(obg_h.at[w], obg_v)
    pltpu.sync_copy(subl_h.at[w], subl_v)
    pltpu.sync_copy(obl_h.at[w], obl_v)
    if pd_h is not None:
        pltpu.sync_copy(pd_h.at[w], pd_v)
    plsc.subcore_barrier()

    def chunk(j, _):
        cidx = q * NCH + j
        cp1 = pltpu.async_copy(q1_h.at[subg_v.at[j]], g1, s1)
        cp3 = pltpu.async_copy(q3_h.at[obg_v.at[j]], g3, s3)
        if pd_h is not None:
            cpp = pltpu.async_copy(t0_h.at[pd_v.at[j]], gp, sp)
        else:
            cpp = pltpu.async_copy(m_h.at[b].at[cidx], gp, sp)
        cp1.wait()
        cp3.wait()
        cpp.wait()

        def comp(r, _):
            for li in range(L // 16):
                sl = pl.ds(li * 16, 16)
                mo[r, sl] = jnp.maximum(g1[r, sl] + g3[r, sl] + gp[r, sl], 0.0)
            return 0
        lax.fori_loop(0, CH, comp, 0)

        cpo = pltpu.async_copy(mo, msg_h.at[b].at[cidx], so)
        pltpu.sync_copy(mo, aggs.at[subl_v.at[j]], add=True)
        pltpu.sync_copy(mo, aggs.at[obl_v.at[j]], add=True)
        cpo.wait()
        return 0
    lax.fori_loop(0, NCH, chunk, 0)

    plsc.subcore_barrier()
    pltpu.sync_copy(aggs.at[pl.ds(base, ZROWS)],
                    agg_h.at[c].at[pl.ds(base, ZROWS)])


def _make_sc_layer(has_pred_gather):
    mesh = plsc.VectorSubcoreMesh(core_axis_name="c", subcore_axis_name="s")
    scratch = [pltpu.VMEM((NCH, CH), jnp.int32) for _ in range(4)]
    if has_pred_gather:
        scratch.append(pltpu.VMEM((NCH, CH), jnp.int32))
    scratch += [pltpu.VMEM((CH, L), _f32) for _ in range(4)]
    scratch += [pltpu.VMEM_SHARED((BPC * NPAD, L), _f32)]
    scratch += [pltpu.SemaphoreType.DMA] * 4
    out_type = (jax.ShapeDtypeStruct((B, CHB, CH, L), _f32),
                jax.ShapeDtypeStruct((NC, BPC * NPAD, L), _f32))

    if has_pred_gather:
        def body(q1_h, q3_h, subg_h, obg_h, subl_h, obl_h, pd_h, t0_h,
                 msg_h, agg_h, subg_v, obg_v, subl_v, obl_v, pd_v,
                 g1, g3, gp, mo, aggs, s1, s3, sp, so):
            _sc_edge_inner(q1_h, q3_h, subg_h, obg_h, subl_h, obl_h, pd_h,
                           t0_h, None, msg_h, agg_h, subg_v, obg_v, subl_v,
                           obl_v, pd_v, g1, g3, gp, mo, aggs, s1, s3, sp, so)
    else:
        def body(q1_h, q3_h, subg_h, obg_h, subl_h, obl_h, m_h,
                 msg_h, agg_h, subg_v, obg_v, subl_v, obl_v,
                 g1, g3, gp, mo, aggs, s1, s3, sp, so):
            _sc_edge_inner(q1_h, q3_h, subg_h, obg_h, subl_h, obl_h, None,
                           None, m_h, msg_h, agg_h, subg_v, obg_v, subl_v,
                           obl_v, None, g1, g3, gp, mo, aggs, s1, s3, sp, so)

    return pl.kernel(body, out_type=out_type, mesh=mesh,
                     scratch_types=scratch)


_sc_layer0 = _make_sc_layer(True)
_sc_layer1 = _make_sc_layer(False)


def kernel(obj_dist, object_feature, rel_ind, pred_dist, obj_table,
           pred_table, W_v, b_v, W_oe, b_oe, W_pe, b_pe, W_e, b_e, W_n,
           W_a, b_n):
    of2 = object_feature.reshape(R, OFS)
    idx2 = obj_dist.reshape(R, 1).astype(jnp.int32)
    bias0 = (b_v + b_oe).reshape(1, L)

    We0, We1 = W_e[0], W_e[1]
    We0a, We0b, We0c = We0[:L], We0[L:2 * L], We0[2 * L:]
    We1a, We1b, We1c = We1[:L], We1[L:2 * L], We1[2 * L:]

    x0, p1_0, p3_0 = _fuse(idx2, of2, obj_table, W_oe, W_v, bias0, We0a, We0c)
    t0 = _t0(pred_table, W_pe, b_pe.reshape(1, L), We0b, b_e[0].reshape(1, L))

    sub = rel_ind[:, :, 0].astype(jnp.int32)
    ob = rel_ind[:, :, 1].astype(jnp.int32)
    boff = jnp.arange(B, dtype=jnp.int32)[:, None]
    subg = (sub + boff * N).reshape(NW, NCH, CH)
    obg = (ob + boff * N).reshape(NW, NCH, CH)
    loff = (boff % BPC) * NPAD
    subl = (sub + loff).reshape(NW, NCH, CH)
    obl = (ob + loff).reshape(NW, NCH, CH)
    pd = pred_dist.astype(jnp.int32).reshape(NW, NCH, CH)

    msg0, agg0p = _sc_layer0(p1_0, p3_0, subg, obg, subl, obl, pd, t0)
    agg0 = agg0p.reshape(NC, BPC, NPAD, L)[:, :, :N, :].reshape(R, L)

    x1, p1_1, p3_1 = _node(x0, agg0, W_n[0], W_a[0], b_n[0].reshape(1, L),
                           We1a, We1c)
    m = _mm(msg0.reshape(B * K, L), We1b,
            b_e[1].reshape(1, L)).reshape(B, CHB, CH, L)

    msg1, agg1p = _sc_layer1(p1_1, p3_1, subg, obg, subl, obl, m)
    agg1 = agg1p.reshape(NC, BPC, NPAD, L)[:, :, :N, :].reshape(R, L)

    x2 = _node(x1, agg1, W_n[1], W_a[1], b_n[1].reshape(1, L))[0]
    return x2.reshape(B, N, L), msg1.reshape(B, K, L)


# speedup vs baseline: 3354.6798x; 1.4988x over previous
import functools

import jax
import jax.numpy as jnp
from jax import lax
from jax.experimental import pallas as pl
from jax.experimental.pallas import tpu as pltpu
from jax.experimental.pallas import tpu_sc as plsc

B, N, K, L, OFS = 8, 1250, 20000, 128, 512
N_OBJ, N_PRED = 150, 50

NC, NS = 2, 16
BPC = B // NC
WPB = NS // BPC
EPW = K // WPB
CH = 100
NCH = EPW // CH
NPAIR = NCH // 2
CH2 = 2 * CH
AROWS = BPC * N
ZR = AROWS // NS
ZTAIL = AROWS - NS * ZR

R = B * N
BR = 400
BE = 2000

_f32 = jnp.float32


def _dot(a, b):
    return jnp.dot(a, b, preferred_element_type=_f32)


def _fuse_body(idx_ref, of_ref, otab_ref, woe_ref, wv_ref, bias0_ref,
               wea_ref, wec_ref, x_ref, p1_ref, p3_ref):
    ot = _dot(otab_ref[...], woe_ref[...])
    iot = lax.broadcasted_iota(jnp.int32, (BR, N_OBJ), 1)
    oh = (idx_ref[...] == iot).astype(_f32)
    emb = _dot(oh, ot)
    x = jnp.maximum(_dot(of_ref[...], wv_ref[...]) + emb + bias0_ref[...], 0.0)
    x_ref[...] = x
    p1_ref[...] = _dot(x, wea_ref[...])
    p3_ref[...] = _dot(x, wec_ref[...])


def _fuse(idx2, of2, obj_table, W_oe, W_v, bias0, Wea, Wec):
    full = lambda s: pl.BlockSpec(s, lambda i: (0, 0))
    return pl.pallas_call(
        _fuse_body,
        grid=(R // BR,),
        in_specs=[
            pl.BlockSpec((BR, 1), lambda i: (i, 0)),
            pl.BlockSpec((BR, OFS), lambda i: (i, 0)),
            full((N_OBJ, L)), full((L, L)), full((OFS, L)), full((1, L)),
            full((L, L)), full((L, L)),
        ],
        out_specs=[pl.BlockSpec((BR, L), lambda i: (i, 0))] * 3,
        out_shape=[jax.ShapeDtypeStruct((R, L), _f32)] * 3,
    )(idx2, of2, obj_table, W_oe, W_v, bias0, Wea, Wec)


def _t0_body(pt_ref, wpe_ref, bpe_ref, web_ref, be0_ref, t0_ref):
    p = _dot(pt_ref[...], wpe_ref[...]) + bpe_ref[...]
    t0_ref[...] = _dot(p, web_ref[...]) + be0_ref[...]


def _t0(pred_table, W_pe, bpe2, We0b, be02):
    return pl.pallas_call(
        _t0_body,
        out_shape=jax.ShapeDtypeStruct((N_PRED, L), _f32),
    )(pred_table, W_pe, bpe2, We0b, be02)


def _mm_body(m_ref, w_ref, b_ref, o_ref):
    o_ref[...] = _dot(m_ref[...], w_ref[...]) + b_ref[...]


def _mm(m2, w, b2):
    full = lambda s: pl.BlockSpec(s, lambda i: (0, 0))
    return pl.pallas_call(
        _mm_body,
        grid=(B * K // BE,),
        in_specs=[pl.BlockSpec((BE, L), lambda i: (i, 0)), full((L, L)),
                  full((1, L))],
        out_specs=pl.BlockSpec((BE, L), lambda i: (i, 0)),
        out_shape=jax.ShapeDtypeStruct((B * K, L), _f32),
    )(m2, w, b2)


NW = NC * NS


def _node_body(x_ref, agg_ref, wn_ref, wa_ref, bn_ref, *rest):
    x = x_ref[...]
    h = _dot(x, wn_ref[...]) + _dot(agg_ref[...], wa_ref[...]) + bn_ref[...]
    xn = x + jnp.maximum(h, 0.0)
    if len(rest) == 1:
        rest[0][...] = xn
    else:
        wea_ref, wec_ref, xo_ref, p1_ref, p3_ref = rest
        xo_ref[...] = xn
        p1_ref[...] = _dot(xn, wea_ref[...])
        p3_ref[...] = _dot(xn, wec_ref[...])


def _node(x, agg, Wn, Wa, bn2, Wea=None, Wec=None):
    full = lambda s: pl.BlockSpec(s, lambda i: (0, 0))
    blk = pl.BlockSpec((BR, L), lambda i: (i, 0))
    n_out = 1 if Wea is None else 3
    ins = [x, agg, Wn, Wa, bn2] + ([] if Wea is None else [Wea, Wec])
    return pl.pallas_call(
        _node_body,
        grid=(R // BR,),
        in_specs=[blk, blk, full((L, L)), full((L, L)), full((1, L))]
        + [full((L, L))] * (n_out - 1),
        out_specs=[blk] * n_out,
        out_shape=[jax.ShapeDtypeStruct((R, L), _f32)] * n_out,
    )(*ins)


def _sc_edge_inner(q1_h, q3_h, subl_h, obl_h, pd_h, t0_h, m_h,
                   msg_h, agg_h, subl_v, obl_v, pd_v,
                   g1, g3, gp, aggs, s1, s3, sp, so):
    c = lax.axis_index("c")
    s = lax.axis_index("s")
    b = c * BPC + s // WPB
    q = s % WPB
    w = b * WPB + q
    base = s * ZR
    e00 = q * EPW

    def zrow(r, _):
        for li in range(L // 16):
            gp[r, pl.ds(li * 16, 16)] = jnp.zeros((16,), _f32)
        return 0
    lax.fori_loop(0, CH2, zrow, 0)
    pltpu.sync_copy(gp, aggs.at[pl.ds(base, CH2)])
    pltpu.sync_copy(gp.at[pl.ds(0, ZR - CH2)],
                    aggs.at[pl.ds(base + CH2, ZR - CH2)])

    @pl.when(s == NS - 1)
    def _():
        pltpu.sync_copy(gp.at[pl.ds(0, ZTAIL)],
                        aggs.at[pl.ds(NS * ZR, ZTAIL)])

    pltpu.sync_copy(subl_h.at[w], subl_v)
    pltpu.sync_copy(obl_h.at[w], obl_v)
    if pd_h is not None:
        pltpu.sync_copy(pd_h.at[w], pd_v)
    plsc.subcore_barrier()

    def pair(t, _):
        j0 = 2 * t
        e0 = e00 + j0 * CH

        @pl.when(t > 0)
        def _():
            pltpu.make_async_copy(gp, msg_h.at[b].at[pl.ds(e0 - CH2, CH2)],
                                  so).wait()

        if pd_h is not None:
            pltpu.async_copy(t0_h.at[w].at[pd_v.at[j0]],
                             gp.at[pl.ds(0, CH)], sp)
            pltpu.async_copy(t0_h.at[w].at[pd_v.at[j0 + 1]],
                             gp.at[pl.ds(CH, CH)], sp)
        else:
            pltpu.async_copy(m_h.at[b].at[pl.ds(e0, CH2)], gp, sp)
        pltpu.async_copy(q1_h.at[c].at[subl_v.at[j0]], g1, s1)
        pltpu.async_copy(q3_h.at[c].at[obl_v.at[j0]], g3, s3)

        pltpu.make_async_copy(q1_h.at[c].at[subl_v.at[j0]], g1, s1).wait()
        pltpu.make_async_copy(q3_h.at[c].at[obl_v.at[j0]], g3, s3).wait()
        if pd_h is not None:
            pltpu.make_async_copy(t0_h.at[w].at[pd_v.at[j0]],
                                  gp.at[pl.ds(0, CH)], sp).wait()
            pltpu.make_async_copy(t0_h.at[w].at[pd_v.at[j0 + 1]],
                                  gp.at[pl.ds(CH, CH)], sp).wait()
        else:
            pltpu.make_async_copy(m_h.at[b].at[pl.ds(e0, CH2)], gp, sp).wait()

        def comp_a(r, _):
            for li in range(L // 16):
                sl = pl.ds(li * 16, 16)
                gp[r, sl] = jnp.maximum(g1[r, sl] + g3[r, sl] + gp[r, sl],
                                        0.0)
            return 0
        lax.fori_loop(0, CH, comp_a, 0)

        pltpu.async_copy(q1_h.at[c].at[subl_v.at[j0 + 1]], g1, s1)
        pltpu.async_copy(q3_h.at[c].at[obl_v.at[j0 + 1]], g3, s3)

        pltpu.sync_copy(gp.at[pl.ds(0, CH)], aggs.at[subl_v.at[j0]], add=True)
        pltpu.sync_copy(gp.at[pl.ds(0, CH)], aggs.at[obl_v.at[j0]], add=True)

        pltpu.make_async_copy(q1_h.at[c].at[subl_v.at[j0 + 1]], g1, s1).wait()
        pltpu.make_async_copy(q3_h.at[c].at[obl_v.at[j0 + 1]], g3, s3).wait()

        def comp_b(r, _):
            for li in range(L // 16):
                sl = pl.ds(li * 16, 16)
                gp[CH + r, sl] = jnp.maximum(
                    g1[r, sl] + g3[r, sl] + gp[CH + r, sl], 0.0)
            return 0
        lax.fori_loop(0, CH, comp_b, 0)

        pltpu.sync_copy(gp.at[pl.ds(CH, CH)], aggs.at[subl_v.at[j0 + 1]],
                        add=True)
        pltpu.sync_copy(gp.at[pl.ds(CH, CH)], aggs.at[obl_v.at[j0 + 1]],
                        add=True)

        pltpu.async_copy(gp, msg_h.at[b].at[pl.ds(e0, CH2)], so)
        return 0
    lax.fori_loop(0, NPAIR, pair, 0)

    pltpu.make_async_copy(gp, msg_h.at[b].at[pl.ds(e00 + (NPAIR - 1) * CH2,
                                                   CH2)], so).wait()
    plsc.subcore_barrier()
    pltpu.sync_copy(aggs.at[pl.ds(base, ZR)], agg_h.at[c].at[pl.ds(base, ZR)])

    @pl.when(s == NS - 1)
    def _():
        pltpu.sync_copy(aggs.at[pl.ds(NS * ZR, ZTAIL)],
                        agg_h.at[c].at[pl.ds(NS * ZR, ZTAIL)])


def _make_sc_layer(has_pred_gather):
    mesh = plsc.VectorSubcoreMesh(core_axis_name="c", subcore_axis_name="s")
    scratch = [pltpu.VMEM((NCH, CH), jnp.int32) for _ in range(2)]
    if has_pred_gather:
        scratch.append(pltpu.VMEM((NCH, CH), jnp.int32))
    scratch += [pltpu.VMEM((CH, L), _f32) for _ in range(2)]
    scratch += [pltpu.VMEM((CH2, L), _f32)]
    scratch += [pltpu.VMEM_SHARED((AROWS, L), _f32)]
    scratch += [pltpu.SemaphoreType.DMA] * 4
    out_type = (jax.ShapeDtypeStruct((B, K, L), _f32),
                jax.ShapeDtypeStruct((NC, AROWS, L), _f32))

    if has_pred_gather:
        def body(q1_h, q3_h, subl_h, obl_h, pd_h, t0_h,
                 msg_h, agg_h, subl_v, obl_v, pd_v,
                 g1, g3, gp, aggs, s1, s3, sp, so):
            _sc_edge_inner(q1_h, q3_h, subl_h, obl_h, pd_h, t0_h, None,
                           msg_h, agg_h, subl_v, obl_v, pd_v,
                           g1, g3, gp, aggs, s1, s3, sp, so)
    else:
        def body(q1_h, q3_h, subl_h, obl_h, m_h,
                 msg_h, agg_h, subl_v, obl_v,
                 g1, g3, gp, aggs, s1, s3, sp, so):
            _sc_edge_inner(q1_h, q3_h, subl_h, obl_h, None, None, m_h,
                           msg_h, agg_h, subl_v, obl_v, None,
                           g1, g3, gp, aggs, s1, s3, sp, so)

    return pl.kernel(body, out_type=out_type, mesh=mesh,
                     scratch_types=scratch)


_sc_layer0 = _make_sc_layer(True)
_sc_layer1 = _make_sc_layer(False)


def kernel(obj_dist, object_feature, rel_ind, pred_dist, obj_table,
           pred_table, W_v, b_v, W_oe, b_oe, W_pe, b_pe, W_e, b_e, W_n,
           W_a, b_n):
    of2 = object_feature.reshape(R, OFS)
    idx2 = obj_dist.reshape(R, 1).astype(jnp.int32)
    bias0 = (b_v + b_oe).reshape(1, L)

    We0, We1 = W_e[0], W_e[1]
    We0a, We0b, We0c = We0[:L], We0[L:2 * L], We0[2 * L:]
    We1a, We1b, We1c = We1[:L], We1[L:2 * L], We1[2 * L:]

    x0, p1_0, p3_0 = _fuse(idx2, of2, obj_table, W_oe, W_v, bias0, We0a, We0c)
    t0 = _t0(pred_table, W_pe, b_pe.reshape(1, L), We0b, b_e[0].reshape(1, L))

    sub = rel_ind[:, :, 0].astype(jnp.int32)
    ob = rel_ind[:, :, 1].astype(jnp.int32)
    boff = jnp.arange(B, dtype=jnp.int32)[:, None]
    loff = (boff % BPC) * N
    subl = (sub + loff).reshape(NW, NCH, CH)
    obl = (ob + loff).reshape(NW, NCH, CH)
    pd = pred_dist.astype(jnp.int32).reshape(NW, NCH, CH)
    t0r = jnp.broadcast_to(t0, (NW, N_PRED, L))

    q1_0 = p1_0.reshape(NC, BPC * N, L)
    q3_0 = p3_0.reshape(NC, BPC * N, L)
    msg0, agg0p = _sc_layer0(q1_0, q3_0, subl, obl, pd, t0r)
    agg0 = agg0p.reshape(R, L)

    x1, p1_1, p3_1 = _node(x0, agg0, W_n[0], W_a[0], b_n[0].reshape(1, L),
                           We1a, We1c)
    m = _mm(msg0.reshape(B * K, L), We1b,
            b_e[1].reshape(1, L)).reshape(B, K, L)

    q1_1 = p1_1.reshape(NC, BPC * N, L)
    q3_1 = p3_1.reshape(NC, BPC * N, L)
    msg1, agg1p = _sc_layer1(q1_1, q3_1, subl, obl, m)
    agg1 = agg1p.reshape(R, L)

    x2 = _node(x1, agg1, W_n[1], W_a[1], b_n[1].reshape(1, L))[0]
    return x2.reshape(B, N, L), msg1
